# Initial kernel scaffold; baseline (speedup 1.0000x reference)
import jax
import jax.numpy as jnp
from jax.experimental import pallas as pl


def kernel(features, idx):
    raise NotImplementedError("write your pallas kernel here")


# speedup vs baseline: 2.3594x; 2.3594x over previous
import functools

import jax
import jax.numpy as jnp
from jax import lax
from jax.experimental import pallas as pl
from jax.experimental.pallas import tpu as pltpu
from jax.experimental.pallas import tpu_sc as plsc

B, C, N = 8, 512, 16384
NPOINT = 4096
NC, NS, L = 2, 16, 16
NW = NC * NS
ROWS = B * C
ROWS_PER_W = ROWS // NW
VPR = NPOINT // L
UNROLL = 8


def _gather_kernel(feat_hbm, idx_hbm, out_hbm, idx_v, row_v, out_v):
    wid = lax.axis_index("s") * NC + lax.axis_index("c")
    b = wid // (NW // B)
    row0 = wid * ROWS_PER_W

    pltpu.sync_copy(idx_hbm.at[b], idx_v)

    def row_body(i, _):
        r = row0 + i
        pltpu.sync_copy(feat_hbm.at[r], row_v)

        def gat_body(j, _):
            base = j * (UNROLL * L)
            for u in range(UNROLL):
                off = base + u * L
                iv = idx_v[pl.ds(off, L)]
                out_v[pl.ds(off, L)] = plsc.load_gather(row_v, [iv])
            return 0

        lax.fori_loop(0, VPR // UNROLL, gat_body, 0)
        pltpu.sync_copy(out_v, out_hbm.at[r])
        return 0

    lax.fori_loop(0, ROWS_PER_W, row_body, 0)


@jax.jit
def _run(feat2d, idx2d):
    mesh = plsc.VectorSubcoreMesh(core_axis_name="c", subcore_axis_name="s")
    f = functools.partial(
        pl.kernel,
        mesh=mesh,
        compiler_params=pltpu.CompilerParams(needs_layout_passes=False),
        out_type=jax.ShapeDtypeStruct((ROWS, NPOINT), jnp.float32),
        scratch_types=[
            pltpu.VMEM((NPOINT,), jnp.int32),
            pltpu.VMEM((N,), jnp.float32),
            pltpu.VMEM((NPOINT,), jnp.float32),
        ],
    )(_gather_kernel)
    return f(feat2d, idx2d)


def kernel(features, idx):
    feat2d = features.reshape(ROWS, N)
    idx2d = idx.astype(jnp.int32)
    out = _run(feat2d, idx2d)
    return out.reshape(B, C, NPOINT)


# speedup vs baseline: 5.1237x; 2.1717x over previous
import functools

import jax
import jax.numpy as jnp
from jax import lax
from jax.experimental import pallas as pl
from jax.experimental.pallas import tpu as pltpu
from jax.experimental.pallas import tpu_sc as plsc

B, C, N = 8, 512, 16384
NPOINT = 4096
NC, NS, L = 2, 16, 16
NW = NC * NS
ROWS = B * C
ROWS_PER_W = ROWS // NW
VPR = NPOINT // L
UNROLL = 8


NB = 4


def _gather_kernel(feat_hbm, idx_hbm, out_hbm, idx_v, *bufs):
    row_bufs = bufs[0:NB]
    out_bufs = bufs[NB:2 * NB]
    row_sems = bufs[2 * NB:3 * NB]
    out_sems = bufs[3 * NB:4 * NB]

    wid = lax.axis_index("s") * NC + lax.axis_index("c")
    b = wid // (NW // B)
    row0 = wid * ROWS_PER_W

    pltpu.sync_copy(idx_hbm.at[b], idx_v)

    for k in range(NB):
        pltpu.async_copy(feat_hbm.at[row0 + k], row_bufs[k], row_sems[k])

    def ring_body(i2, _):
        for k in range(NB):
            i = i2 * NB + k
            r = row0 + i
            pltpu.make_async_copy(feat_hbm.at[r], row_bufs[k], row_sems[k]).wait()

            @pl.when(i2 > 0)
            def _():
                pltpu.make_async_copy(out_bufs[k], out_hbm.at[r - NB],
                                      out_sems[k]).wait()

            def gat_body(j, _):
                base = j * (UNROLL * L)
                for u in range(UNROLL):
                    off = base + u * L
                    iv = idx_v[pl.ds(off, L)]
                    out_bufs[k][pl.ds(off, L)] = plsc.load_gather(row_bufs[k], [iv])
                return 0

            lax.fori_loop(0, VPR // UNROLL, gat_body, 0)

            pltpu.async_copy(out_bufs[k], out_hbm.at[r], out_sems[k])

            @pl.when(i + NB < ROWS_PER_W)
            def _():
                pltpu.async_copy(feat_hbm.at[r + NB], row_bufs[k], row_sems[k])
        return 0

    lax.fori_loop(0, ROWS_PER_W // NB, ring_body, 0)

    for k in range(NB):
        r = row0 + ROWS_PER_W - NB + k
        pltpu.make_async_copy(out_bufs[k], out_hbm.at[r], out_sems[k]).wait()


@jax.jit
def _run(feat2d, idx2d):
    mesh = plsc.VectorSubcoreMesh(core_axis_name="c", subcore_axis_name="s")
    f = functools.partial(
        pl.kernel,
        mesh=mesh,
        compiler_params=pltpu.CompilerParams(needs_layout_passes=False),
        out_type=jax.ShapeDtypeStruct((ROWS, NPOINT), jnp.float32),
        scratch_types=[
            pltpu.VMEM((NPOINT,), jnp.int32),
            *[pltpu.VMEM((N,), jnp.float32) for _ in range(NB)],
            *[pltpu.VMEM((NPOINT,), jnp.float32) for _ in range(NB)],
            *[pltpu.SemaphoreType.DMA for _ in range(2 * NB)],
        ],
    )(_gather_kernel)
    return f(feat2d, idx2d)


def kernel(features, idx):
    feat2d = features.reshape(ROWS, N)
    idx2d = idx.astype(jnp.int32)
    out = _run(feat2d, idx2d)
    return out.reshape(B, C, NPOINT)
